# baseline (speedup 1.0000x reference)
import numpy as np
import jax
import jax.numpy as jnp
from jax import lax
from jax.experimental import pallas as pl
from jax.experimental.pallas import tpu as pltpu
from jax.experimental.pallas import tpu_sc as plsc

IN_DIM = 256
OUT_DIM = 256
GRID_SIZE = 16

_KNOTS = np.linspace(-1.0, 1.0, GRID_SIZE).astype(np.float32)
_INV_H = np.zeros((GRID_SIZE, 1), np.float32)
_INV_H[:-1, 0] = 1.0 / (_KNOTS[1:] - _KNOTS[:-1] + 1e-8)
_SCALE = np.float32((GRID_SIZE - 1) / 2.0)

_SC_ROWS = 65536
_NC, _NS, _L = 2, 16, 16
_NW = _NC * _NS


def _tc_fused_kernel(x_ref, vt_ref, invh_ref, knots_ref, wt_ref, b_ref, o_ref):
    vt = vt_ref[...]
    vt1 = pltpu.roll(vt, shift=GRID_SIZE - 1, axis=0)
    slope = (vt1 - vt) * invh_ref[...]
    icept = vt - knots_ref[...] * slope

    xb = jnp.clip(x_ref[...], -1.0, 1.0)
    u = jnp.clip((xb + 1.0) * _SCALE, 0.0, float(GRID_SIZE - 2))
    lo = u < 8.0
    s = u.astype(jnp.int32)
    s7 = jnp.bitwise_and(s, 7)
    a = jnp.where(
        lo,
        jnp.take_along_axis(icept[0:8, :], s7, axis=0, mode="promise_in_bounds"),
        jnp.take_along_axis(icept[8:16, :], s7, axis=0, mode="promise_in_bounds"),
    )
    b = jnp.where(
        lo,
        jnp.take_along_axis(slope[0:8, :], s7, axis=0, mode="promise_in_bounds"),
        jnp.take_along_axis(slope[8:16, :], s7, axis=0, mode="promise_in_bounds"),
    )
    acc = a + b * xb

    o_ref[...] = (
        jnp.dot(acc, wt_ref[...], preferred_element_type=jnp.float32)
        + b_ref[...]
    )


def _tc_matmul_kernel(y_ref, wt_ref, b_ref, o_ref):
    o_ref[...] = (
        jnp.dot(y_ref[...], wt_ref[...], preferred_element_type=jnp.float32)
        + b_ref[...]
    )


def _tc_fused(x, vt, invh, knots, wt, bias, tb):
    nb, d = x.shape
    return pl.pallas_call(
        _tc_fused_kernel,
        grid=(nb // tb,),
        in_specs=[
            pl.BlockSpec((tb, d), lambda i: (i, 0)),
            pl.BlockSpec((GRID_SIZE, d), lambda i: (0, 0)),
            pl.BlockSpec((GRID_SIZE, 1), lambda i: (0, 0)),
            pl.BlockSpec((GRID_SIZE, 1), lambda i: (0, 0)),
            pl.BlockSpec((d, OUT_DIM), lambda i: (0, 0)),
            pl.BlockSpec((1, OUT_DIM), lambda i: (0, 0)),
        ],
        out_specs=pl.BlockSpec((tb, OUT_DIM), lambda i: (i, 0)),
        out_shape=jax.ShapeDtypeStruct((nb, OUT_DIM), jnp.float32),
        compiler_params=pltpu.CompilerParams(
            dimension_semantics=("parallel",),
        ),
    )(x, vt, invh, knots, wt, bias)


def _tc_matmul(y, wt, bias, tb):
    nb, d = y.shape
    return pl.pallas_call(
        _tc_matmul_kernel,
        grid=(nb // tb,),
        in_specs=[
            pl.BlockSpec((tb, d), lambda i: (i, 0)),
            pl.BlockSpec((d, OUT_DIM), lambda i: (0, 0)),
            pl.BlockSpec((1, OUT_DIM), lambda i: (0, 0)),
        ],
        out_specs=pl.BlockSpec((tb, OUT_DIM), lambda i: (i, 0)),
        out_shape=jax.ShapeDtypeStruct((nb, OUT_DIM), jnp.float32),
        compiler_params=pltpu.CompilerParams(
            dimension_semantics=("parallel",),
        ),
    )(y, wt, bias)


def _sc_interp(x_flat, icept_flat, slope_flat):
    n = x_flat.shape[0]
    per_w = n // _NW
    ch = min(32768, per_w)
    nchunk = per_w // ch
    nvec = ch // _L

    mesh = plsc.VectorSubcoreMesh(core_axis_name="c", subcore_axis_name="s")

    def body(x_hbm, ic_hbm, sl_hbm, y_hbm, xv, yv, ic_v, sl_v):
        wid = lax.axis_index("s") * _NC + lax.axis_index("c")
        pltpu.sync_copy(ic_hbm, ic_v)
        pltpu.sync_copy(sl_hbm, sl_v)
        diota = lax.iota(jnp.int32, _L) * GRID_SIZE
        base = wid * per_w

        def chunk_body(k, carry):
            off = base + k * ch
            pltpu.sync_copy(x_hbm.at[pl.ds(off, ch)], xv)

            def vec_body(i, c2):
                x16 = xv[pl.ds(i * _L, _L)]
                xc = jnp.minimum(jnp.maximum(x16, -1.0), 1.0)
                u = (xc + 1.0) * _SCALE
                si = jnp.minimum(u.astype(jnp.int32), GRID_SIZE - 2)
                dblk = lax.rem(i, IN_DIM // _L) * (_L * GRID_SIZE)
                idx = (diota + dblk) + si
                a = plsc.load_gather(ic_v, [idx])
                b = plsc.load_gather(sl_v, [idx])
                yv[pl.ds(i * _L, _L)] = a + b * xc
                return c2

            lax.fori_loop(0, nvec, vec_body, 0)
            pltpu.sync_copy(yv, y_hbm.at[pl.ds(off, ch)])
            return carry

        lax.fori_loop(0, nchunk, chunk_body, 0)

    f = pl.kernel(
        body,
        out_type=jax.ShapeDtypeStruct((n,), jnp.float32),
        mesh=mesh,
        compiler_params=pltpu.CompilerParams(needs_layout_passes=False),
        scratch_types=[
            pltpu.VMEM((ch,), jnp.float32),
            pltpu.VMEM((ch,), jnp.float32),
            pltpu.VMEM((IN_DIM * GRID_SIZE,), jnp.float32),
            pltpu.VMEM((IN_DIM * GRID_SIZE,), jnp.float32),
        ],
    )
    return f(x_flat, icept_flat, slope_flat)


@jax.jit
def kernel(x, values, mix_w, mix_b):
    B, D = x.shape
    vt = values.T
    wt = mix_w.T
    bias = mix_b.reshape(1, OUT_DIM)
    invh = jnp.asarray(_INV_H)
    knots = jnp.asarray(_KNOTS[:, None])

    outs = []
    tc_rows = B - _SC_ROWS
    if tc_rows:
        outs.append(_tc_fused(x[:tc_rows], vt, invh, knots, wt, bias,
                              min(8192, tc_rows)))
    if _SC_ROWS:
        slope_t = (vt[1:, :] - vt[:-1, :]) * invh[:-1]
        icept_t = vt[:-1, :] - knots[:-1] * slope_t
        pad = jnp.zeros((1, D), jnp.float32)
        slope_f = jnp.concatenate([slope_t, pad], 0).T.reshape(-1)
        icept_f = jnp.concatenate([icept_t, pad], 0).T.reshape(-1)
        y = _sc_interp(x[tc_rows:].reshape(-1), icept_f, slope_f)
        outs.append(_tc_matmul(y.reshape(_SC_ROWS, D), wt, bias, 8192))
    return outs[0] if len(outs) == 1 else jnp.concatenate(outs, axis=0)


# speedup vs baseline: 5.0509x; 5.0509x over previous
import numpy as np
import jax
import jax.numpy as jnp
from jax import lax
from jax.experimental import pallas as pl
from jax.experimental.pallas import tpu as pltpu
from jax.experimental.pallas import tpu_sc as plsc

IN_DIM = 256
OUT_DIM = 256
GRID_SIZE = 16

_KNOTS = np.linspace(-1.0, 1.0, GRID_SIZE).astype(np.float32)
_INV_H = np.zeros((GRID_SIZE, 1), np.float32)
_INV_H[:-1, 0] = 1.0 / (_KNOTS[1:] - _KNOTS[:-1] + 1e-8)
_SCALE = np.float32((GRID_SIZE - 1) / 2.0)

_SC_ROWS = 8192
_NC, _NS, _L = 2, 16, 16
_NW = _NC * _NS


def _tc_fused_kernel(x_ref, vt_ref, invh_ref, knots_ref, wt_ref, b_ref, o_ref):
    vt = vt_ref[...]
    vt1 = pltpu.roll(vt, shift=GRID_SIZE - 1, axis=0)
    slope = (vt1 - vt) * invh_ref[...]
    icept = vt - knots_ref[...] * slope

    xb = jnp.clip(x_ref[...], -1.0, 1.0)
    u = jnp.clip((xb + 1.0) * _SCALE, 0.0, float(GRID_SIZE - 2))
    lo = u < 8.0
    s = u.astype(jnp.int32)
    s7 = jnp.bitwise_and(s, 7)
    a = jnp.where(
        lo,
        jnp.take_along_axis(icept[0:8, :], s7, axis=0, mode="promise_in_bounds"),
        jnp.take_along_axis(icept[8:16, :], s7, axis=0, mode="promise_in_bounds"),
    )
    b = jnp.where(
        lo,
        jnp.take_along_axis(slope[0:8, :], s7, axis=0, mode="promise_in_bounds"),
        jnp.take_along_axis(slope[8:16, :], s7, axis=0, mode="promise_in_bounds"),
    )
    acc = a + b * xb

    o_ref[...] = (
        jnp.dot(acc, wt_ref[...], preferred_element_type=jnp.float32)
        + b_ref[...]
    )


def _tc_matmul_kernel(y_ref, wt_ref, b_ref, o_ref):
    o_ref[...] = (
        jnp.dot(y_ref[...], wt_ref[...], preferred_element_type=jnp.float32)
        + b_ref[...]
    )


def _tc_fused(x, vt, invh, knots, wt, bias, tb):
    nb, d = x.shape
    return pl.pallas_call(
        _tc_fused_kernel,
        grid=(nb // tb,),
        in_specs=[
            pl.BlockSpec((tb, d), lambda i: (i, 0)),
            pl.BlockSpec((GRID_SIZE, d), lambda i: (0, 0)),
            pl.BlockSpec((GRID_SIZE, 1), lambda i: (0, 0)),
            pl.BlockSpec((GRID_SIZE, 1), lambda i: (0, 0)),
            pl.BlockSpec((d, OUT_DIM), lambda i: (0, 0)),
            pl.BlockSpec((1, OUT_DIM), lambda i: (0, 0)),
        ],
        out_specs=pl.BlockSpec((tb, OUT_DIM), lambda i: (i, 0)),
        out_shape=jax.ShapeDtypeStruct((nb, OUT_DIM), jnp.float32),
        compiler_params=pltpu.CompilerParams(
            dimension_semantics=("parallel",),
        ),
    )(x, vt, invh, knots, wt, bias)


def _tc_matmul(y, wt, bias, tb):
    nb, d = y.shape
    return pl.pallas_call(
        _tc_matmul_kernel,
        grid=(nb // tb,),
        in_specs=[
            pl.BlockSpec((tb, d), lambda i: (i, 0)),
            pl.BlockSpec((d, OUT_DIM), lambda i: (0, 0)),
            pl.BlockSpec((1, OUT_DIM), lambda i: (0, 0)),
        ],
        out_specs=pl.BlockSpec((tb, OUT_DIM), lambda i: (i, 0)),
        out_shape=jax.ShapeDtypeStruct((nb, OUT_DIM), jnp.float32),
        compiler_params=pltpu.CompilerParams(
            dimension_semantics=("parallel",),
        ),
    )(y, wt, bias)


def _sc_interp(x_flat, icept_flat, slope_flat):
    n = x_flat.shape[0]
    per_w = n // _NW
    ch = min(32768, per_w)
    nchunk = per_w // ch
    nvec = ch // _L

    mesh = plsc.VectorSubcoreMesh(core_axis_name="c", subcore_axis_name="s")

    def body(x_hbm, ic_hbm, sl_hbm, y_hbm, xv, yv, ic_v, sl_v):
        wid = lax.axis_index("s") * _NC + lax.axis_index("c")
        pltpu.sync_copy(ic_hbm, ic_v)
        pltpu.sync_copy(sl_hbm, sl_v)
        diota = lax.iota(jnp.int32, _L) * GRID_SIZE
        base = wid * per_w

        rows = ch // IN_DIM
        ndv = IN_DIM // _L

        def chunk_body(k, carry):
            off = base + k * ch
            pltpu.sync_copy(x_hbm.at[pl.ds(off, ch)], xv)

            @plsc.parallel_loop(0, rows, unroll=4)
            def _row(r):
                rbase = r * IN_DIM
                for dv in range(ndv):
                    x16 = xv[pl.ds(rbase + dv * _L, _L)]
                    xc = jnp.minimum(jnp.maximum(x16, -1.0), 1.0)
                    u = (xc + 1.0) * _SCALE
                    si = jnp.minimum(u.astype(jnp.int32), GRID_SIZE - 2)
                    idx = diota + si
                    tslc = pl.ds(dv * _L * GRID_SIZE, _L * GRID_SIZE)
                    a = plsc.load_gather(ic_v.at[tslc], [idx])
                    b = plsc.load_gather(sl_v.at[tslc], [idx])
                    yv[pl.ds(rbase + dv * _L, _L)] = a + b * xc

            pltpu.sync_copy(yv, y_hbm.at[pl.ds(off, ch)])
            return carry

        lax.fori_loop(0, nchunk, chunk_body, 0)

    f = pl.kernel(
        body,
        out_type=jax.ShapeDtypeStruct((n,), jnp.float32),
        mesh=mesh,
        compiler_params=pltpu.CompilerParams(needs_layout_passes=False),
        scratch_types=[
            pltpu.VMEM((ch,), jnp.float32),
            pltpu.VMEM((ch,), jnp.float32),
            pltpu.VMEM((IN_DIM * GRID_SIZE,), jnp.float32),
            pltpu.VMEM((IN_DIM * GRID_SIZE,), jnp.float32),
        ],
    )
    return f(x_flat, icept_flat, slope_flat)


@jax.jit
def kernel(x, values, mix_w, mix_b):
    B, D = x.shape
    vt = values.T
    wt = mix_w.T
    bias = mix_b.reshape(1, OUT_DIM)
    invh = jnp.asarray(_INV_H)
    knots = jnp.asarray(_KNOTS[:, None])

    outs = []
    tc_rows = B - _SC_ROWS
    if tc_rows:
        outs.append(_tc_fused(x[:tc_rows], vt, invh, knots, wt, bias,
                              min(8192, tc_rows)))
    if _SC_ROWS:
        slope_t = (vt[1:, :] - vt[:-1, :]) * invh[:-1]
        icept_t = vt[:-1, :] - knots[:-1] * slope_t
        pad = jnp.zeros((1, D), jnp.float32)
        slope_f = jnp.concatenate([slope_t, pad], 0).T.reshape(-1)
        icept_f = jnp.concatenate([icept_t, pad], 0).T.reshape(-1)
        y = _sc_interp(x[tc_rows:].reshape(-1), icept_f, slope_f)
        outs.append(_tc_matmul(y.reshape(_SC_ROWS, D), wt, bias, 8192))
    return outs[0] if len(outs) == 1 else jnp.concatenate(outs, axis=0)


# speedup vs baseline: 17.4575x; 3.4563x over previous
import numpy as np
import jax
import jax.numpy as jnp
from jax.experimental import pallas as pl
from jax.experimental.pallas import tpu as pltpu

IN_DIM = 256
OUT_DIM = 256
GRID_SIZE = 16

_KNOTS = np.linspace(-1.0, 1.0, GRID_SIZE).astype(np.float32)
_INV_H = np.zeros((GRID_SIZE, 1), np.float32)
_INV_H[:-1, 0] = 1.0 / (_KNOTS[1:] - _KNOTS[:-1] + 1e-8)
_SCALE = np.float32((GRID_SIZE - 1) / 2.0)


def _kan_kernel(x_ref, vt_ref, invh_ref, knots_ref, wt_ref, b_ref, o_ref):
    vt = vt_ref[...]
    vt1 = pltpu.roll(vt, shift=GRID_SIZE - 1, axis=0)
    slope = (vt1 - vt) * invh_ref[...]
    icept = vt - knots_ref[...] * slope

    xb = jnp.clip(x_ref[...], -1.0, 1.0)
    u = (xb + 1.0) * _SCALE
    lo = u < 8.0
    s = u.astype(jnp.int32)
    s7 = jnp.bitwise_and(s, 7)
    a = jnp.where(
        lo,
        jnp.take_along_axis(icept[0:8, :], s7, axis=0, mode="promise_in_bounds"),
        jnp.take_along_axis(icept[8:16, :], s7, axis=0, mode="promise_in_bounds"),
    )
    b = jnp.where(
        lo,
        jnp.take_along_axis(slope[0:8, :], s7, axis=0, mode="promise_in_bounds"),
        jnp.take_along_axis(slope[8:16, :], s7, axis=0, mode="promise_in_bounds"),
    )
    acc = a + b * xb

    o_ref[...] = (
        jnp.dot(acc, wt_ref[...], preferred_element_type=jnp.float32)
        + b_ref[...]
    )


@jax.jit
def kernel(x, values, mix_w, mix_b):
    B, D = x.shape
    TB = min(8192, B)
    grid = (B // TB,)
    vt = values.T
    wt = mix_w.T
    bias = mix_b.reshape(1, OUT_DIM)
    invh = jnp.asarray(_INV_H)
    knots = jnp.asarray(_KNOTS[:, None])
    return pl.pallas_call(
        _kan_kernel,
        grid=grid,
        in_specs=[
            pl.BlockSpec((TB, D), lambda i: (i, 0)),
            pl.BlockSpec((GRID_SIZE, D), lambda i: (0, 0)),
            pl.BlockSpec((GRID_SIZE, 1), lambda i: (0, 0)),
            pl.BlockSpec((GRID_SIZE, 1), lambda i: (0, 0)),
            pl.BlockSpec((D, OUT_DIM), lambda i: (0, 0)),
            pl.BlockSpec((1, OUT_DIM), lambda i: (0, 0)),
        ],
        out_specs=pl.BlockSpec((TB, OUT_DIM), lambda i: (i, 0)),
        out_shape=jax.ShapeDtypeStruct((B, OUT_DIM), jnp.float32),
        compiler_params=pltpu.CompilerParams(
            dimension_semantics=("parallel",),
        ),
    )(x, vt, invh, knots, wt, bias)
